# baseline (speedup 1.0000x reference)
import jax
import jax.numpy as jnp
from jax import lax
from jax.experimental import pallas as pl
from jax.experimental.pallas import tpu as pltpu
from jax.experimental.pallas import tpu_sc as plsc

MARGIN = 1.0
N = 4096
NPROP = 128
L = 16
NC, NS = 2, 16
NW = NC * NS
PPW = NPROP // NW
NB = 2 * PPW
NCHUNK = N // L
CAP = N + 3 * L
NEG_PAD = -1.0e30
NBLK = 8
BLK = N // NBLK


def _sc_body(packed_hbm, out_hbm, packed_v, part_v, off_v, big_v, sem):
    wid = lax.axis_index("c") * NS + lax.axis_index("s")

    copies = []
    for j in range(NBLK):
        blk = (wid + j) % NBLK
        sl = pl.ds(blk * BLK, BLK)
        copies.append(pltpu.async_copy(packed_hbm.at[sl], packed_v.at[sl],
                                       sem))
    for c in copies:
        c.wait()

    off_v[pl.ds(0, L)] = jnp.zeros((L,), jnp.int32)
    off_v[pl.ds(L, L)] = jnp.zeros((L,), jnp.int32)

    lane = lax.broadcasted_iota(jnp.int32, (L,), 0)

    def chunk_body(k, carry):
        w = packed_v[pl.ds(k * L, L)]
        key = w & 255
        mine = (key >> 3) == wid
        t_idx = key & (NB - 1)
        s = plsc.bitcast(w & ~255, jnp.float32)
        rank, last = plsc.scan_count(key, mask=mine)
        base = plsc.load_gather(off_v, [t_idx])
        addr = t_idx * CAP + base + (rank - 1)
        plsc.store_scatter(big_v, [addr], s, mask=mine)
        plsc.addupdate_scatter(off_v, [t_idx], rank, mask=last & mine)
        return carry

    lax.fori_loop(0, NCHUNK, chunk_body, jnp.int32(0))

    pad_vec = jnp.full((L,), NEG_PAD, jnp.float32)

    def t_body(t, carry):
        acc0, pairs = carry
        offs = off_v[pl.ds(2 * t, L)]
        nneg, npos = offs[0], offs[1]
        negbase = (2 * t) * CAP
        posbase = negbase + CAP
        big_v[pl.ds(negbase + nneg, L)] = pad_vec
        pairs = pairs + npos * nneg
        nch = (nneg + (L - 1)) // L

        @plsc.parallel_loop(0, npos, carry=acc0)
        def acc(i, a):
            coef = MARGIN - big_v[pl.ds(posbase + i, L)][0]

            def neg_body(c, aa):
                nv = big_v[pl.ds(negbase + c * L, L)]
                return aa + jnp.maximum(coef + nv, 0.0)

            return lax.fori_loop(0, nch, neg_body, a)

        return acc, pairs

    acc, pairs = lax.fori_loop(
        0, PPW, t_body, (jnp.zeros((L,), jnp.float32), jnp.int32(0)))

    loss = jnp.sum(acc)
    part = jnp.where(lane == 0, loss,
                     jnp.where(lane == 1, pairs.astype(jnp.float32), 0.0))
    part_v[...] = part
    pltpu.sync_copy(part_v, out_hbm.at[wid])


@jax.jit
def _pairwise_loss_sc(packed):
    mesh = plsc.VectorSubcoreMesh(core_axis_name="c", subcore_axis_name="s")
    scratch = [
        pltpu.VMEM((N,), jnp.int32),
        pltpu.VMEM((L,), jnp.float32),
        pltpu.VMEM((2 * L,), jnp.int32),
        pltpu.VMEM((NB * CAP,), jnp.float32),
        pltpu.SemaphoreType.DMA,
    ]
    parts = pl.kernel(
        _sc_body,
        out_type=jax.ShapeDtypeStruct((NW, L), jnp.float32),
        mesh=mesh,
        scratch_types=scratch,
        compiler_params=pltpu.CompilerParams(needs_layout_passes=False),
    )(packed)
    loss = parts[:, 0].sum()
    pairs = parts[:, 1].sum()
    return jnp.where(pairs == 0.0, 0.0, loss / jnp.maximum(pairs, 1.0))


def kernel(scores, labels, property_ids):
    scores = scores.reshape(-1).astype(jnp.float32)
    labels = labels.reshape(-1).astype(jnp.int32)
    props = property_ids.reshape(-1).astype(jnp.int32)
    scores_i = lax.bitcast_convert_type(scores, jnp.int32)
    packed = (scores_i & ~255) | (props << 1) | labels
    return _pairwise_loss_sc(packed)


# speedup vs baseline: 1.0664x; 1.0664x over previous
import jax
import jax.numpy as jnp
from jax import lax
from jax.experimental import pallas as pl
from jax.experimental.pallas import tpu as pltpu
from jax.experimental.pallas import tpu_sc as plsc

MARGIN = 1.0
N = 4096
NPROP = 128
L = 16
NC, NS = 2, 16
NW = NC * NS
PPW = NPROP // NW
NB = 2 * PPW
NCHUNK = N // L
CAP = N + 3 * L
NEG_PAD = -1.0e30
NBLK = 16
BLK = N // NBLK


def _sc_body(packed_hbm, out_hbm, packed_v, part_v, off_v, big_v, sem):
    wid = lax.axis_index("c") * NS + lax.axis_index("s")

    copies = []
    for j in range(NBLK):
        blk = (wid + j) % NBLK
        sl = pl.ds(blk * BLK, BLK)
        copies.append(pltpu.async_copy(packed_hbm.at[sl], packed_v.at[sl],
                                       sem))
    for c in copies:
        c.wait()

    off_v[pl.ds(0, L)] = jnp.zeros((L,), jnp.int32)
    off_v[pl.ds(L, L)] = jnp.zeros((L,), jnp.int32)

    lane = lax.broadcasted_iota(jnp.int32, (L,), 0)

    @plsc.parallel_loop(0, NCHUNK, carry=jnp.zeros((L,), jnp.int32),
                        unroll=1)
    def offs_vec(k, offs_vec):
        w = packed_v[pl.ds(k * L, L)]
        key = w & 255
        mine = (key >> 3) == wid
        t_idx = key & (NB - 1)
        s = plsc.bitcast(w & ~255, jnp.float32)
        rank, _ = plsc.scan_count(key, mask=mine)
        base = lax.gather(
            offs_vec, t_idx[:, None],
            lax.GatherDimensionNumbers(
                offset_dims=(), collapsed_slice_dims=(0,),
                start_index_map=(0,)),
            slice_sizes=(1,),
            mode=lax.GatherScatterMode.PROMISE_IN_BOUNDS)
        addr = t_idx * CAP + base + (rank - 1)
        plsc.store_scatter(big_v, [addr], s, mask=mine)
        delta = jnp.zeros((L,), jnp.int32)
        for b in range(NB):
            cb = plsc.all_reduce_population_count(mine & (t_idx == b))
            delta = jnp.where(lane == b, delta + cb, delta)
        return offs_vec + delta

    off_v[pl.ds(0, L)] = offs_vec

    pad_vec = jnp.full((L,), NEG_PAD, jnp.float32)

    def t_body(t, carry):
        acc0, pairs = carry
        offs = off_v[pl.ds(2 * t, L)]
        nneg, npos = offs[0], offs[1]
        negbase = (2 * t) * CAP
        posbase = negbase + CAP
        for j in range(3):
            big_v[pl.ds(negbase + nneg + j * L, L)] = pad_vec
        pairs = pairs + npos * nneg
        nch = (nneg + (L - 1)) // L

        @plsc.parallel_loop(0, npos, carry=acc0, unroll=2)
        def acc(i, a):
            coef = MARGIN - big_v[pl.ds(posbase + i, L)][0]
            for j in range(3):
                nv = big_v[pl.ds(negbase + j * L, L)]
                a = a + jnp.maximum(coef + nv, 0.0)

            def neg_body(c, aa):
                nv = big_v[pl.ds(negbase + c * L, L)]
                return aa + jnp.maximum(coef + nv, 0.0)

            return lax.fori_loop(3, nch, neg_body, a)

        return acc, pairs

    acc, pairs = lax.fori_loop(
        0, PPW, t_body, (jnp.zeros((L,), jnp.float32), jnp.int32(0)))

    loss = jnp.sum(acc)
    part = jnp.where(lane == 0, loss,
                     jnp.where(lane == 1, pairs.astype(jnp.float32), 0.0))
    part_v[...] = part
    pltpu.sync_copy(part_v, out_hbm.at[wid])


@jax.jit
def _pairwise_loss_sc(packed):
    mesh = plsc.VectorSubcoreMesh(core_axis_name="c", subcore_axis_name="s")
    scratch = [
        pltpu.VMEM((N,), jnp.int32),
        pltpu.VMEM((L,), jnp.float32),
        pltpu.VMEM((2 * L,), jnp.int32),
        pltpu.VMEM((NB * CAP,), jnp.float32),
        pltpu.SemaphoreType.DMA,
    ]
    parts = pl.kernel(
        _sc_body,
        out_type=jax.ShapeDtypeStruct((NW, L), jnp.float32),
        mesh=mesh,
        scratch_types=scratch,
        compiler_params=pltpu.CompilerParams(needs_layout_passes=False),
    )(packed)
    loss = parts[:, 0].sum()
    pairs = parts[:, 1].sum()
    return jnp.where(pairs == 0.0, 0.0, loss / jnp.maximum(pairs, 1.0))


def kernel(scores, labels, property_ids):
    scores = scores.reshape(-1).astype(jnp.float32)
    labels = labels.reshape(-1).astype(jnp.int32)
    props = property_ids.reshape(-1).astype(jnp.int32)
    scores_i = lax.bitcast_convert_type(scores, jnp.int32)
    packed = (scores_i & ~255) | (props << 1) | labels
    return _pairwise_loss_sc(packed)
